# baseline (speedup 1.0000x reference)
import functools

import jax
import jax.numpy as jnp
from jax import lax
from jax.experimental import pallas as pl
from jax.experimental.pallas import tpu as pltpu
from jax.experimental.pallas import tpu_sc as plsc

_NC = 2
_NS = 16
_NW = _NC * _NS
_R = 64


def _gather_fn(table, x, fields):
    batch, padded_fields = x.shape
    vocab, emb_dim = table.shape
    mesh = plsc.VectorSubcoreMesh(core_axis_name="c", subcore_axis_name="s")
    rows_per_w = batch // _NW
    steps = rows_per_w // _R

    _FP = 32
    block_bytes = _R * _FP * emb_dim * jnp.dtype(table.dtype).itemsize

    @functools.partial(
        pl.kernel,
        out_type=jax.ShapeDtypeStruct((batch, fields, emb_dim), table.dtype),
        mesh=mesh,
        compiler_params=pltpu.CompilerParams(use_tc_tiling_on_sc=False),
        scratch_types=[
            pltpu.VMEM((_R, _FP), jnp.int32),
            pltpu.VMEM((_R, _FP, emb_dim), table.dtype),
            pltpu.SemaphoreType.DMA,
        ],
    )
    def gather_kernel(table_hbm, idx_hbm, out_hbm, idx_v, rows_v, sem):
        wid = lax.axis_index("s") * _NC + lax.axis_index("c")
        base = wid * rows_per_w

        @pl.loop(0, steps)
        def _(i):
            row0 = base + i * _R
            pltpu.sync_copy(
                idx_hbm.at[pl.ds(row0, _R), pl.ds(0, _FP)], idx_v
            )

            @pl.loop(0, _R)
            def _(r):
                pltpu.async_copy(table_hbm.at[idx_v.at[r]], rows_v.at[r], sem)

            @pl.loop(0, _R)
            def _(r):
                pltpu.make_async_copy(
                    table_hbm.at[idx_v.at[r]], rows_v.at[r], sem
                ).wait()
            pltpu.sync_copy(
                rows_v.at[:, pl.ds(0, fields), :], out_hbm.at[pl.ds(row0, _R)]
            )

    return gather_kernel(table, x)


def kernel(x, embedding):
    batch, fields = x.shape
    xp = jnp.pad(x, ((0, 0), (0, 128 - fields)))
    return _gather_fn(embedding, xp, fields)


# speedup vs baseline: 2.2668x; 2.2668x over previous
import functools

import jax
import jax.numpy as jnp
from jax import lax
from jax.experimental import pallas as pl
from jax.experimental.pallas import tpu as pltpu
from jax.experimental.pallas import tpu_sc as plsc

_NC = 2
_NS = 16
_NW = _NC * _NS
_R = 64


def _gather_fn(table, x):
    batch, fields = x.shape
    vocab, emb_dim = table.shape
    mesh = plsc.VectorSubcoreMesh(core_axis_name="c", subcore_axis_name="s")
    rows_per_w = batch // _NW
    steps = rows_per_w // _R

    @functools.partial(
        pl.kernel,
        out_type=jax.ShapeDtypeStruct((batch, fields, emb_dim), table.dtype),
        mesh=mesh,
        compiler_params=pltpu.CompilerParams(use_tc_tiling_on_sc=False),
        scratch_types=[
            pltpu.VMEM((_R, fields), jnp.int32),
            pltpu.VMEM((_R, fields, emb_dim), table.dtype),
            pltpu.SemaphoreType.DMA,
        ],
    )
    def gather_kernel(table_hbm, idx_hbm, out_hbm, idx_v, rows_v, sem):
        wid = lax.axis_index("s") * _NC + lax.axis_index("c")
        base = wid * rows_per_w

        @pl.loop(0, steps)
        def _(i):
            row0 = base + i * _R
            pltpu.sync_copy(idx_hbm.at[pl.ds(row0, _R)], idx_v)

            @pl.loop(0, _R)
            def _(r):
                pltpu.async_copy(table_hbm.at[idx_v.at[r]], rows_v.at[r], sem)

            pltpu.make_async_copy(
                out_hbm.at[pl.ds(row0, _R)], rows_v, sem
            ).wait()
            pltpu.sync_copy(rows_v, out_hbm.at[pl.ds(row0, _R)])

    return gather_kernel(table, x)


def kernel(x, embedding):
    return _gather_fn(embedding, x)


# speedup vs baseline: 2.2966x; 1.0132x over previous
import functools

import jax
import jax.numpy as jnp
from jax import lax
from jax.experimental import pallas as pl
from jax.experimental.pallas import tpu as pltpu
from jax.experimental.pallas import tpu_sc as plsc

_NC = 2
_NS = 16
_NW = _NC * _NS
_C = 128
_BLK = 1024
_G = _BLK // _C


def _gather_fn(table, xflat):
    _, emb = table.shape
    per_w = xflat.shape[1]
    steps = per_w // _BLK
    mesh = plsc.VectorSubcoreMesh(core_axis_name="c", subcore_axis_name="s")

    @functools.partial(
        pl.kernel,
        out_type=jax.ShapeDtypeStruct((_NW, steps, _BLK, emb), table.dtype),
        mesh=mesh,
        compiler_params=pltpu.CompilerParams(use_tc_tiling_on_sc=False),
        scratch_types=[
            pltpu.VMEM((per_w,), jnp.int32),
            pltpu.VMEM((2, _BLK, emb), table.dtype),
            pltpu.SemaphoreType.DMA,
            pltpu.SemaphoreType.DMA,
            pltpu.SemaphoreType.DMA,
        ],
    )
    def gather_kernel(table_hbm, idx_hbm, out_hbm, idx_v, rows_v, g0, g1, wsem):
        wid = lax.axis_index("s") * _NC + lax.axis_index("c")
        pltpu.sync_copy(idx_hbm.at[wid], idx_v)
        gsems = (g0, g1)

        def issue(s, buf):
            for c in range(_G):
                pltpu.async_copy(
                    table_hbm.at[idx_v.at[pl.ds(s * _BLK + c * _C, _C)]],
                    rows_v.at[buf, pl.ds(c * _C, _C)],
                    gsems[buf],
                )

        def drain(buf, sem):
            pltpu.make_async_copy(out_hbm.at[0, 0], rows_v.at[buf], sem).wait()

        issue(0, 0)
        for s in range(steps):
            cur = s % 2
            nxt = 1 - cur
            if s + 1 < steps:
                if s >= 1:
                    drain(nxt, wsem)
                issue(s + 1, nxt)
            drain(cur, gsems[cur])
            pltpu.async_copy(rows_v.at[cur], out_hbm.at[wid, s], wsem)
        drain(0, wsem)
        drain(1, wsem)

    return gather_kernel(table, xflat)


def kernel(x, embedding):
    batch, fields = x.shape
    emb = embedding.shape[1]
    flat = _gather_fn(embedding, x.reshape(_NW, -1))
    return flat.reshape(batch, fields, emb)
